# baseline (speedup 1.0000x reference)
import functools

import jax
import jax.numpy as jnp
from jax import lax
from jax.experimental import pallas as pl
from jax.experimental.pallas import tpu as pltpu
from jax.experimental.pallas import tpu_sc as plsc

_BATCH = 16384
_HIST = 200
_EMBED = 16
_VOCAB = 1000000
_NC = 2
_NS = 16
_NW = _NC * _NS
_ROWS_PER_W = _BATCH // _NW
_GROUP = 128
_NGROUP = _ROWS_PER_W // _GROUP
_LB = _HIST // 8
_GV = _HIST * _GROUP

_TBLK = 16384
_TGRID = (_VOCAB + _TBLK - 1) // _TBLK


def _proj_body(bs_ref, x_ref, w_ref, o_ref):
    o_ref[...] = jnp.sum(x_ref[...] * w_ref[...], axis=0) + bs_ref[0]


def _project(tableT, w2d, bs):
    return pl.pallas_call(
        _proj_body,
        grid=(_TGRID,),
        in_specs=[
            pl.BlockSpec(memory_space=pltpu.SMEM),
            pl.BlockSpec((_EMBED, _TBLK), lambda i: (0, i)),
            pl.BlockSpec((_EMBED, 1), lambda i: (0, 0)),
        ],
        out_specs=pl.BlockSpec((_TBLK,), lambda i: (i,)),
        out_shape=jax.ShapeDtypeStruct((_VOCAB,), jnp.float32),
    )(bs, tableT, w2d)


_LBA = 13
_LBB = _LB - _LBA


def _sc_body(xp_hbm, t_hbm, out_hbm, t_sh,
             idx_v, valA, valB, out_v, sem_i, sem_gA, sem_gB):
    sid = lax.axis_index("s")
    wid = sid * _NC + lax.axis_index("c")

    def issue_idx(g):
        pltpu.async_copy(
            xp_hbm.at[:, pl.ds(wid * _NGROUP + g, 1), :, :], idx_v, sem_i)

    issue_idx(0)
    stage = _VOCAB // 8

    @pl.when(sid < 8)
    def _():
        pltpu.sync_copy(t_hbm.at[pl.ds(sid * stage, stage)],
                        t_sh.at[pl.ds(sid * stage, stage)])
    plsc.subcore_barrier()

    def wait_idx():
        pltpu.make_async_copy(
            xp_hbm.at[:, pl.ds(0, 1), :, :], idx_v, sem_i).wait()

    def issue_gathers(lb0, nlb, vbuf, sem):
        def body(lb, c):
            for dl in range(8):
                pltpu.async_copy(
                    t_sh.at[idx_v.at[lb0 + lb, 0, dl]],
                    vbuf.at[pl.ds((lb * 8 + dl) * _GROUP, _GROUP)],
                    sem)
            return c
        lax.fori_loop(0, nlb, body, 0)

    def wait_gathers(nlb, vbuf, sem):
        pltpu.make_async_copy(
            t_hbm.at[pl.ds(0, nlb * 8 * _GROUP)], vbuf, sem).wait()

    def accumulate(accs, nlb, vbuf):
        def lbody(l, accs):
            return tuple(
                accs[u] + vbuf[pl.ds(l * _GROUP + u * 16, 16)]
                for u in range(8))
        return lax.fori_loop(0, nlb * 8, lbody, accs)

    for g in range(_NGROUP):
        wait_idx()
        issue_gathers(0, _LBA, valA, sem_gA)
        issue_gathers(_LBA, _LBB, valB, sem_gB)
        zeros = tuple(jnp.zeros((16,), jnp.float32) for _ in range(8))
        wait_gathers(_LBA, valA, sem_gA)
        accs = accumulate(zeros, _LBA, valA)
        wait_gathers(_LBB, valB, sem_gB)
        if g + 1 < _NGROUP:
            issue_idx(g + 1)
        accs = accumulate(accs, _LBB, valB)
        for u in range(8):
            y = 1.0 / (1.0 + jnp.exp(-accs[u]))
            y = ((y * 10000.0 + 0.5).astype(jnp.int32).astype(jnp.float32)
                 * 1e-4)
            out_v[pl.ds(g * _GROUP + u * 16, 16)] = y
    pltpu.sync_copy(out_v, out_hbm.at[pl.ds(wid * _ROWS_PER_W, _ROWS_PER_W)])


@jax.jit
def _launch(xp, tableT, w2d, bs):
    t = _project(tableT, w2d, bs)
    mesh = plsc.VectorSubcoreMesh(core_axis_name="c", subcore_axis_name="s")
    f = functools.partial(
        pl.kernel,
        out_type=jax.ShapeDtypeStruct((_BATCH,), jnp.float32),
        mesh=mesh,
        compiler_params=pltpu.CompilerParams(use_tc_tiling_on_sc=False),
        scratch_types=[
            pltpu.VMEM_SHARED((_VOCAB,), jnp.float32),
            pltpu.VMEM((_LB, 1, 8, _GROUP), jnp.int32),
            pltpu.VMEM((_LBA * 8 * _GROUP,), jnp.float32),
            pltpu.VMEM((_LBB * 8 * _GROUP,), jnp.float32),
            pltpu.VMEM((_ROWS_PER_W,), jnp.float32),
            pltpu.SemaphoreType.DMA,
            pltpu.SemaphoreType.DMA,
            pltpu.SemaphoreType.DMA,
        ],
    )(_sc_body)
    return f(xp, t)


def kernel(x, table, W, b):
    xp = (x.astype(jnp.int32).T
          .reshape(_LB, 8, _BATCH // _GROUP, _GROUP)
          .transpose(0, 2, 1, 3))
    w2d = (W.astype(jnp.float32) / float(_HIST)).reshape(_EMBED, 1)
    bs = (b.astype(jnp.float32) / float(_HIST)).reshape(1)
    out = _launch(xp, table.T, w2d, bs)
    return out.reshape(_BATCH, 1)


# speedup vs baseline: 1.2856x; 1.2856x over previous
import functools

import jax
import jax.numpy as jnp
from jax import lax
from jax.experimental import pallas as pl
from jax.experimental.pallas import tpu as pltpu
from jax.experimental.pallas import tpu_sc as plsc

_BATCH = 16384
_HIST = 200
_EMBED = 16
_VOCAB = 1000000
_NC = 2
_NS = 16
_NW = _NC * _NS
_ROWS_PER_W = _BATCH // _NW
_GROUP = 128
_NGROUP = _ROWS_PER_W // _GROUP
_LB = _HIST // 8
_GV = _HIST * _GROUP

_TBLK = 131072
_TGRID = (_VOCAB + _TBLK - 1) // _TBLK


def _proj_body(bs_ref, x_ref, w_ref, o_ref):
    o_ref[...] = jnp.sum(x_ref[...] * w_ref[...], axis=0) + bs_ref[0]


def _project(tableT, w2d, bs):
    return pl.pallas_call(
        _proj_body,
        grid=(_TGRID,),
        in_specs=[
            pl.BlockSpec(memory_space=pltpu.SMEM),
            pl.BlockSpec((_EMBED, _TBLK), lambda i: (0, i)),
            pl.BlockSpec((_EMBED, 1), lambda i: (0, 0)),
        ],
        out_specs=pl.BlockSpec((_TBLK,), lambda i: (i,)),
        out_shape=jax.ShapeDtypeStruct((_VOCAB,), jnp.float32),
    )(bs, tableT, w2d)


_LBA = 13
_LBB = _LB - _LBA


def _sc_body(xp_hbm, t_hbm, out_hbm, t_sh,
             idx_v, valA, valB, out_v, sem_i, sem_gA, sem_gB):
    sid = lax.axis_index("s")
    wid = sid * _NC + lax.axis_index("c")

    def issue_idx(g):
        pltpu.async_copy(
            xp_hbm.at[:, pl.ds(wid * _NGROUP + g, 1), :, :], idx_v, sem_i)

    issue_idx(0)
    stage = _VOCAB // 8

    @pl.when(sid < 8)
    def _():
        pltpu.sync_copy(t_hbm.at[pl.ds(sid * stage, stage)],
                        t_sh.at[pl.ds(sid * stage, stage)])
    plsc.subcore_barrier()

    def wait_idx():
        pltpu.make_async_copy(
            xp_hbm.at[:, pl.ds(0, 1), :, :], idx_v, sem_i).wait()

    def issue_gathers(lb0, nlb, vbuf, sem):
        def body(lb, c):
            for dl in range(8):
                pltpu.async_copy(
                    t_sh.at[idx_v.at[lb0 + lb, 0, dl]],
                    vbuf.at[pl.ds((lb * 8 + dl) * _GROUP, _GROUP)],
                    sem)
            return c
        lax.fori_loop(0, nlb, body, 0)

    def wait_gathers(nlb, vbuf, sem):
        pltpu.make_async_copy(
            t_hbm.at[pl.ds(0, nlb * 8 * _GROUP)], vbuf, sem).wait()

    def accumulate(accs, nlb, vbuf):
        def lbody(l, accs):
            return tuple(
                accs[u] + vbuf[pl.ds(l * _GROUP + u * 16, 16)]
                for u in range(8))
        return lax.fori_loop(0, nlb * 8, lbody, accs)

    for g in range(_NGROUP):
        wait_idx()
        issue_gathers(0, _LBA, valA, sem_gA)
        issue_gathers(_LBA, _LBB, valB, sem_gB)
        zeros = tuple(jnp.zeros((16,), jnp.float32) for _ in range(8))
        wait_gathers(_LBA, valA, sem_gA)
        accs = accumulate(zeros, _LBA, valA)
        wait_gathers(_LBB, valB, sem_gB)
        if g + 1 < _NGROUP:
            issue_idx(g + 1)
        accs = accumulate(accs, _LBB, valB)
        for u in range(8):
            y = 1.0 / (1.0 + jnp.exp(-accs[u]))
            y = ((y * 10000.0 + 0.5).astype(jnp.int32).astype(jnp.float32)
                 * 1e-4)
            out_v[pl.ds(g * _GROUP + u * 16, 16)] = y
    pltpu.sync_copy(out_v, out_hbm.at[pl.ds(wid * _ROWS_PER_W, _ROWS_PER_W)])


@jax.jit
def _launch(xp, tableT, w2d, bs):
    t = _project(tableT, w2d, bs)
    mesh = plsc.VectorSubcoreMesh(core_axis_name="c", subcore_axis_name="s")
    f = functools.partial(
        pl.kernel,
        out_type=jax.ShapeDtypeStruct((_BATCH,), jnp.float32),
        mesh=mesh,
        compiler_params=pltpu.CompilerParams(use_tc_tiling_on_sc=False),
        scratch_types=[
            pltpu.VMEM_SHARED((_VOCAB,), jnp.float32),
            pltpu.VMEM((_LB, 1, 8, _GROUP), jnp.int32),
            pltpu.VMEM((_LBA * 8 * _GROUP,), jnp.float32),
            pltpu.VMEM((_LBB * 8 * _GROUP,), jnp.float32),
            pltpu.VMEM((_ROWS_PER_W,), jnp.float32),
            pltpu.SemaphoreType.DMA,
            pltpu.SemaphoreType.DMA,
            pltpu.SemaphoreType.DMA,
        ],
    )(_sc_body)
    return f(xp, t)


def kernel(x, table, W, b):
    xp = (x.astype(jnp.int32).T
          .reshape(_LB, 8, _BATCH // _GROUP, _GROUP)
          .transpose(0, 2, 1, 3))
    w2d = (W.astype(jnp.float32) / float(_HIST)).reshape(_EMBED, 1)
    bs = (b.astype(jnp.float32) / float(_HIST)).reshape(1)
    out = _launch(xp, table.T, w2d, bs)
    return out.reshape(_BATCH, 1)


# speedup vs baseline: 1.2897x; 1.0032x over previous
import functools

import jax
import jax.numpy as jnp
from jax import lax
from jax.experimental import pallas as pl
from jax.experimental.pallas import tpu as pltpu
from jax.experimental.pallas import tpu_sc as plsc

_BATCH = 16384
_HIST = 200
_EMBED = 16
_VOCAB = 1000000
_NC = 2
_NS = 16
_NW = _NC * _NS
_ROWS_PER_W = _BATCH // _NW
_GROUP = 128
_NGROUP = _ROWS_PER_W // _GROUP
_LB = _HIST // 8
_GV = _HIST * _GROUP

_TBLK = 262144
_TGRID = (_VOCAB + _TBLK - 1) // _TBLK


def _proj_body(bs_ref, x_ref, w_ref, o_ref):
    o_ref[...] = jnp.sum(x_ref[...] * w_ref[...], axis=0) + bs_ref[0]


def _project(tableT, w2d, bs):
    return pl.pallas_call(
        _proj_body,
        grid=(_TGRID,),
        in_specs=[
            pl.BlockSpec(memory_space=pltpu.SMEM),
            pl.BlockSpec((_EMBED, _TBLK), lambda i: (0, i)),
            pl.BlockSpec((_EMBED, 1), lambda i: (0, 0)),
        ],
        out_specs=pl.BlockSpec((_TBLK,), lambda i: (i,)),
        out_shape=jax.ShapeDtypeStruct((_VOCAB,), jnp.float32),
    )(bs, tableT, w2d)


_LBA = 13
_LBB = _LB - _LBA


def _sc_body(xp_hbm, t_hbm, out_hbm, t_sh,
             idx_v, valA, valB, out_v, sem_i, sem_gA, sem_gB):
    sid = lax.axis_index("s")
    wid = sid * _NC + lax.axis_index("c")

    def issue_idx(g):
        pltpu.async_copy(
            xp_hbm.at[:, pl.ds(wid * _NGROUP + g, 1), :, :], idx_v, sem_i)

    issue_idx(0)
    stage = _VOCAB // 8

    @pl.when(sid < 8)
    def _():
        pltpu.sync_copy(t_hbm.at[pl.ds(sid * stage, stage)],
                        t_sh.at[pl.ds(sid * stage, stage)])
    plsc.subcore_barrier()

    def wait_idx():
        pltpu.make_async_copy(
            xp_hbm.at[:, pl.ds(0, 1), :, :], idx_v, sem_i).wait()

    def issue_gathers(lb0, nlb, vbuf, sem):
        def body(lb, c):
            for dl in range(8):
                pltpu.async_copy(
                    t_sh.at[idx_v.at[lb0 + lb, 0, dl]],
                    vbuf.at[pl.ds((lb * 8 + dl) * _GROUP, _GROUP)],
                    sem)
            return c
        lax.fori_loop(0, nlb, body, 0)

    def wait_gathers(nlb, vbuf, sem):
        pltpu.make_async_copy(
            t_hbm.at[pl.ds(0, nlb * 8 * _GROUP)], vbuf, sem).wait()

    def accumulate(accs, nlb, vbuf):
        def lbody(l, accs):
            return tuple(
                accs[u] + vbuf[pl.ds(l * _GROUP + u * 16, 16)]
                for u in range(8))
        return lax.fori_loop(0, nlb * 8, lbody, accs)

    for g in range(_NGROUP):
        wait_idx()
        issue_gathers(0, _LBA, valA, sem_gA)
        issue_gathers(_LBA, _LBB, valB, sem_gB)
        zeros = tuple(jnp.zeros((16,), jnp.float32) for _ in range(8))
        wait_gathers(_LBA, valA, sem_gA)
        accs = accumulate(zeros, _LBA, valA)
        wait_gathers(_LBB, valB, sem_gB)
        if g + 1 < _NGROUP:
            issue_idx(g + 1)
        accs = accumulate(accs, _LBB, valB)
        for u in range(8):
            y = 1.0 / (1.0 + jnp.exp(-accs[u]))
            y = ((y * 10000.0 + 0.5).astype(jnp.int32).astype(jnp.float32)
                 * 1e-4)
            out_v[pl.ds(g * _GROUP + u * 16, 16)] = y
    pltpu.sync_copy(out_v, out_hbm.at[pl.ds(wid * _ROWS_PER_W, _ROWS_PER_W)])


@jax.jit
def _launch(xp, tableT, w2d, bs):
    t = _project(tableT, w2d, bs)
    mesh = plsc.VectorSubcoreMesh(core_axis_name="c", subcore_axis_name="s")
    f = functools.partial(
        pl.kernel,
        out_type=jax.ShapeDtypeStruct((_BATCH,), jnp.float32),
        mesh=mesh,
        compiler_params=pltpu.CompilerParams(use_tc_tiling_on_sc=False),
        scratch_types=[
            pltpu.VMEM_SHARED((_VOCAB,), jnp.float32),
            pltpu.VMEM((_LB, 1, 8, _GROUP), jnp.int32),
            pltpu.VMEM((_LBA * 8 * _GROUP,), jnp.float32),
            pltpu.VMEM((_LBB * 8 * _GROUP,), jnp.float32),
            pltpu.VMEM((_ROWS_PER_W,), jnp.float32),
            pltpu.SemaphoreType.DMA,
            pltpu.SemaphoreType.DMA,
            pltpu.SemaphoreType.DMA,
        ],
    )(_sc_body)
    return f(xp, t)


def kernel(x, table, W, b):
    xp = (x.astype(jnp.int32).T
          .reshape(_LB, 8, _BATCH // _GROUP, _GROUP)
          .transpose(0, 2, 1, 3))
    w2d = (W.astype(jnp.float32) / float(_HIST)).reshape(_EMBED, 1)
    bs = (b.astype(jnp.float32) / float(_HIST)).reshape(1)
    out = _launch(xp, table.T, w2d, bs)
    return out.reshape(_BATCH, 1)
